# baseline (speedup 1.0000x reference)
import functools

import jax
import jax.numpy as jnp
from jax import lax
from jax.experimental import pallas as pl
from jax.experimental.pallas import tpu as pltpu
from jax.experimental.pallas import tpu_sc as plsc

_EPS = 1e-6
_K = 16
_T2 = 256
_T = 64

_DN = (((1,), (1,)), ((), ()))


def _mm(x, w):
    return lax.dot_general(x, w, _DN, preferred_element_type=jnp.float32)


def _knn_body(qpt_ref, qp_ref, idx_ref):
    b = pl.program_id(0)
    n = qpt_ref.shape[2]
    t = qp_ref.shape[1]
    qa = qpt_ref[0]
    qt = qp_ref[0]
    mm = lax.dot_general(qt, qa, (((1,), (0,)), ((), ())),
                         preferred_element_type=jnp.float32)
    sqx = jnp.sum(qt * qt, axis=1, keepdims=True)
    sqy = jnp.sum(qa * qa, axis=0, keepdims=True)
    neg = -((-2.0) * mm + sqx + sqy)
    lane = lax.broadcasted_iota(jnp.int32, (t, n), 1)
    kio = lax.broadcasted_iota(jnp.int32, (t, _K), 1)
    acc = jnp.zeros((t, _K), jnp.int32)
    for j in range(_K):
        m = jnp.max(neg, axis=1, keepdims=True)
        cand = jnp.where(neg == m, lane, n)
        am = jnp.min(cand, axis=1, keepdims=True)
        acc = jnp.where(kio == j, am + b * n, acc)
        neg = jnp.where(lane == am, -jnp.inf, neg)
    idx_ref[0] = acc


def _knn(q_pos, q_post):
    b, n, _ = q_pos.shape
    return pl.pallas_call(
        _knn_body,
        grid=(b, n // _T2),
        in_specs=[
            pl.BlockSpec((1, 3, n), lambda bi, i: (bi, 0, 0)),
            pl.BlockSpec((1, _T2, 3), lambda bi, i: (bi, i, 0)),
        ],
        out_specs=pl.BlockSpec((1, _T2, _K), lambda bi, i: (bi, i, 0)),
        out_shape=jax.ShapeDtypeStruct((b, n, _K), jnp.int32),
    )(q_post, q_pos)


def _sc_gather(qtab, gidx2d):
    rows = gidx2d.shape[0] * gidx2d.shape[1]
    dcol = qtab.shape[1]
    nw = 32
    per_w = rows // nw
    ch = 128
    nch = per_w // ch
    mesh = plsc.VectorSubcoreMesh(core_axis_name="c", subcore_axis_name="s")

    @functools.partial(
        pl.kernel,
        out_type=jax.ShapeDtypeStruct((rows, dcol), jnp.float32),
        mesh=mesh,
        scratch_types=[
            pltpu.VMEM((nch, ch), jnp.int32),
            pltpu.VMEM((ch, dcol), jnp.float32),
            pltpu.VMEM((ch, dcol), jnp.float32),
            pltpu.SemaphoreType.DMA,
            pltpu.SemaphoreType.DMA,
        ],
    )
    def gk(qtab_hbm, gidx_hbm, out_hbm, idx_v, buf0, buf1, sem0, sem1):
        wid = lax.axis_index("s") * 2 + lax.axis_index("c")
        pltpu.sync_copy(gidx_hbm.at[pl.ds(wid * nch, nch)], idx_v)
        bufs = (buf0, buf1)
        sems = (sem0, sem1)
        cps = [None, None]
        cps[0] = pltpu.async_copy(qtab_hbm.at[idx_v.at[0]], buf0, sem0)
        for i in range(nch):
            cur = i % 2
            if i + 1 < nch:
                cps[1 - cur] = pltpu.async_copy(
                    qtab_hbm.at[idx_v.at[i + 1]], bufs[1 - cur], sems[1 - cur])
            cps[cur].wait()
            pltpu.sync_copy(bufs[cur], out_hbm.at[pl.ds((wid * nch + i) * ch, ch)])

    return gk(qtab, gidx2d)


def _main_body(qg_ref, qt_ref, wk_ref, wa_ref, wp_ref, o_ref):
    t = qt_ref.shape[1]
    c = wa_ref.shape[0]
    r = t * _K
    h = []
    for d in range(3):
        lv = qg_ref[0, :, d, :]
        qrep = jnp.broadcast_to(
            qt_ref[0, :, d, :][:, None, :], (t, _K, c)).reshape(r, c)
        fe = jnp.concatenate([lv - qrep, qrep], axis=1)
        h.append(_mm(fe, wk_ref[...]))
    dm = [_mm(h[d], wa_ref[...]) for d in range(3)]
    dnorm = jnp.sqrt(dm[0] * dm[0] + dm[1] * dm[1] + dm[2] * dm[2]) + _EPS
    inv = 1.0 / dnorm
    ddir = [dm[d] * inv for d in range(3)]
    dot = h[0] * ddir[0] + h[1] * ddir[1] + h[2] * ddir[2]
    acted = jnp.where(dot >= 0, dot, 0.2 * dot)
    ha = [h[d] + (acted - dot) * ddir[d] for d in range(3)]
    dp = [_mm(ha[d], wp_ref[...]) for d in range(3)]
    dotp = (ha[0] * dp[0] + ha[1] * dp[1] + ha[2] * dp[2]).reshape(t, _K, c)
    m = jnp.max(dotp, axis=1)
    ha3 = [x.reshape(t, _K, c) for x in ha]
    outd = [x[:, _K - 1, :] for x in ha3]
    for k in range(_K - 2, -1, -1):
        sel = dotp[:, k, :] == m
        outd = [jnp.where(sel, ha3[d][:, k, :], outd[d]) for d in range(3)]
    o_ref[0] = jnp.stack(outd, axis=1)


def _main(qg, qt, wk, wa, wp):
    b, nk, _, c = qg.shape
    n = qt.shape[1]
    r = _T * _K
    return pl.pallas_call(
        _main_body,
        grid=(b, n // _T),
        in_specs=[
            pl.BlockSpec((1, r, 3, c), lambda bi, i: (bi, i, 0, 0)),
            pl.BlockSpec((1, _T, 3, c), lambda bi, i: (bi, i, 0, 0)),
            pl.BlockSpec((c, 2 * c), lambda bi, i: (0, 0)),
            pl.BlockSpec((c, c), lambda bi, i: (0, 0)),
            pl.BlockSpec((c, c), lambda bi, i: (0, 0)),
        ],
        out_specs=pl.BlockSpec((1, _T, 3, c), lambda bi, i: (bi, i, 0, 0)),
        out_shape=jax.ShapeDtypeStruct((b, n, 3, c), jnp.float32),
    )(qg, qt, wk, wa, wp)


def kernel(q, q_pos, W_knn, W_act, W_pool):
    b, n, c, _ = q.shape
    qtab = jnp.transpose(q, (0, 1, 3, 2))
    q_post = jnp.transpose(q_pos, (0, 2, 1))
    gidx = _knn(q_pos, q_post)
    table = qtab.reshape(b * n, 3 * c)
    outs = []
    for bi in range(b):
        qg = _sc_gather(table, gidx[bi].reshape(n * _K // 128, 128))
        outs.append(_main(qg.reshape(1, n * _K, 3, c), qtab[bi:bi + 1],
                          W_knn, W_act, W_pool))
    out = jnp.concatenate(outs, axis=0)
    return jnp.transpose(out, (0, 1, 3, 2))


# speedup vs baseline: 1.1618x; 1.1618x over previous
import functools

import jax
import jax.numpy as jnp
from jax import lax
from jax.experimental import pallas as pl
from jax.experimental.pallas import tpu as pltpu
from jax.experimental.pallas import tpu_sc as plsc

_EPS = 1e-6
_K = 16
_T2 = 256
_T = 64

_DN = (((1,), (1,)), ((), ()))


def _mm(x, w):
    return lax.dot_general(x, w, _DN, preferred_element_type=jnp.float32)


def _knn_body(qpt_ref, qp_ref, idx_ref):
    b = pl.program_id(0)
    n = qpt_ref.shape[2]
    t = qp_ref.shape[1]
    qa = qpt_ref[0]
    qt = qp_ref[0]
    mm = lax.dot_general(qt, qa, (((1,), (0,)), ((), ())),
                         preferred_element_type=jnp.float32)
    sqx = jnp.sum(qt * qt, axis=1, keepdims=True)
    sqy = jnp.sum(qa * qa, axis=0, keepdims=True)
    neg = -((-2.0) * mm + sqx + sqy)
    lane = lax.broadcasted_iota(jnp.int32, (t, n), 1)
    kio = lax.broadcasted_iota(jnp.int32, (t, _K), 1)
    acc = jnp.zeros((t, _K), jnp.int32)
    for j in range(_K):
        m = jnp.max(neg, axis=1, keepdims=True)
        cand = jnp.where(neg == m, lane, n)
        am = jnp.min(cand, axis=1, keepdims=True)
        acc = jnp.where(kio == j, am + b * n, acc)
        neg = jnp.where(lane == am, -jnp.inf, neg)
    idx_ref[0] = acc


def _knn(q_pos, q_post):
    b, n, _ = q_pos.shape
    return pl.pallas_call(
        _knn_body,
        grid=(b, n // _T2),
        in_specs=[
            pl.BlockSpec((1, 3, n), lambda bi, i: (bi, 0, 0)),
            pl.BlockSpec((1, _T2, 3), lambda bi, i: (bi, i, 0)),
        ],
        out_specs=pl.BlockSpec((1, _T2, _K), lambda bi, i: (bi, i, 0)),
        out_shape=jax.ShapeDtypeStruct((b, n, _K), jnp.int32),
    )(q_post, q_pos)


def _sc_gather(qtab, gidx2d):
    rows = gidx2d.shape[0] * gidx2d.shape[1]
    dcol = qtab.shape[1]
    nw = 32
    per_w = rows // nw
    ch = 128
    nch = per_w // ch
    mesh = plsc.VectorSubcoreMesh(core_axis_name="c", subcore_axis_name="s")

    @functools.partial(
        pl.kernel,
        out_type=jax.ShapeDtypeStruct((rows, dcol), jnp.float32),
        mesh=mesh,
        scratch_types=[
            pltpu.VMEM((nch, ch), jnp.int32),
            pltpu.VMEM((ch, dcol), jnp.float32),
            pltpu.VMEM((ch, dcol), jnp.float32),
            pltpu.SemaphoreType.DMA,
            pltpu.SemaphoreType.DMA,
        ],
    )
    def gk(qtab_hbm, gidx_hbm, out_hbm, idx_v, buf0, buf1, sem0, sem1):
        wid = lax.axis_index("s") * 2 + lax.axis_index("c")
        pltpu.sync_copy(gidx_hbm.at[pl.ds(wid * nch, nch)], idx_v)
        bufs = (buf0, buf1)
        sems = (sem0, sem1)
        cps = [None, None]
        cps[0] = pltpu.async_copy(qtab_hbm.at[idx_v.at[0]], buf0, sem0)
        for i in range(nch):
            cur = i % 2
            if i + 1 < nch:
                cps[1 - cur] = pltpu.async_copy(
                    qtab_hbm.at[idx_v.at[i + 1]], bufs[1 - cur], sems[1 - cur])
            cps[cur].wait()
            pltpu.sync_copy(bufs[cur], out_hbm.at[pl.ds((wid * nch + i) * ch, ch)])

    return gk(qtab, gidx2d)


def _main_body(qg_ref, qt_ref, wk_ref, wa_ref, wp_ref, o_ref):
    t = qt_ref.shape[1]
    c = wa_ref.shape[0]
    r = t * _K
    h = []
    for d in range(3):
        lv = qg_ref[0, :, d, :]
        qrep = jnp.broadcast_to(
            qt_ref[0, :, d, :][:, None, :], (t, _K, c)).reshape(r, c)
        fe = jnp.concatenate([lv - qrep, qrep], axis=1)
        h.append(_mm(fe, wk_ref[...]))
    dm = [_mm(h[d], wa_ref[...]) for d in range(3)]
    dnorm = jnp.sqrt(dm[0] * dm[0] + dm[1] * dm[1] + dm[2] * dm[2]) + _EPS
    inv = 1.0 / dnorm
    ddir = [dm[d] * inv for d in range(3)]
    dot = h[0] * ddir[0] + h[1] * ddir[1] + h[2] * ddir[2]
    acted = jnp.where(dot >= 0, dot, 0.2 * dot)
    ha = [h[d] + (acted - dot) * ddir[d] for d in range(3)]
    dp = [_mm(ha[d], wp_ref[...]) for d in range(3)]
    dotp = (ha[0] * dp[0] + ha[1] * dp[1] + ha[2] * dp[2]).reshape(t, _K, c)
    m = jnp.max(dotp, axis=1)
    ha3 = [x.reshape(t, _K, c) for x in ha]
    outd = [x[:, _K - 1, :] for x in ha3]
    for k in range(_K - 2, -1, -1):
        sel = dotp[:, k, :] == m
        outd = [jnp.where(sel, ha3[d][:, k, :], outd[d]) for d in range(3)]
    o_ref[0] = jnp.stack(outd, axis=1)


def _main(qg, qt, wk, wa, wp):
    b, nk, _, c = qg.shape
    n = qt.shape[1]
    r = _T * _K
    return pl.pallas_call(
        _main_body,
        grid=(b, n // _T),
        in_specs=[
            pl.BlockSpec((1, r, 3, c), lambda bi, i: (bi, i, 0, 0)),
            pl.BlockSpec((1, _T, 3, c), lambda bi, i: (bi, i, 0, 0)),
            pl.BlockSpec((c, 2 * c), lambda bi, i: (0, 0)),
            pl.BlockSpec((c, c), lambda bi, i: (0, 0)),
            pl.BlockSpec((c, c), lambda bi, i: (0, 0)),
        ],
        out_specs=pl.BlockSpec((1, _T, 3, c), lambda bi, i: (bi, i, 0, 0)),
        out_shape=jax.ShapeDtypeStruct((b, n, 3, c), jnp.float32),
    )(qg, qt, wk, wa, wp)


def kernel(q, q_pos, W_knn, W_act, W_pool):
    b, n, c, _ = q.shape
    qtab = jnp.transpose(q, (0, 1, 3, 2))
    q_post = jnp.transpose(q_pos, (0, 2, 1))
    gidx = _knn(q_pos, q_post)
    qg = _sc_gather(qtab.reshape(b * n, 3 * c),
                    gidx.reshape(b * n * _K // 128, 128))
    out = _main(qg.reshape(b, n * _K, 3, c), qtab, W_knn, W_act, W_pool)
    return jnp.transpose(out, (0, 1, 3, 2))


# speedup vs baseline: 1.1700x; 1.0071x over previous
import functools

import jax
import jax.numpy as jnp
from jax import lax
from jax.experimental import pallas as pl
from jax.experimental.pallas import tpu as pltpu
from jax.experimental.pallas import tpu_sc as plsc

_EPS = 1e-6
_K = 16
_T2 = 256
_T = 128

_DN = (((1,), (1,)), ((), ()))


def _mm(x, w):
    return lax.dot_general(x, w, _DN, preferred_element_type=jnp.float32)


def _knn_body(qpt_ref, qp_ref, idx_ref):
    b = pl.program_id(0)
    n = qpt_ref.shape[2]
    t = qp_ref.shape[1]
    qa = qpt_ref[0]
    qt = qp_ref[0]
    mm = lax.dot_general(qt, qa, (((1,), (0,)), ((), ())),
                         preferred_element_type=jnp.float32)
    sqx = jnp.sum(qt * qt, axis=1, keepdims=True)
    sqy = jnp.sum(qa * qa, axis=0, keepdims=True)
    neg = -((-2.0) * mm + sqx + sqy)
    lane = lax.broadcasted_iota(jnp.int32, (t, n), 1)
    kio = lax.broadcasted_iota(jnp.int32, (t, _K), 1)
    acc = jnp.zeros((t, _K), jnp.int32)
    for j in range(_K):
        m = jnp.max(neg, axis=1, keepdims=True)
        cand = jnp.where(neg == m, lane, n)
        am = jnp.min(cand, axis=1, keepdims=True)
        acc = jnp.where(kio == j, am + b * n, acc)
        neg = jnp.where(lane == am, -jnp.inf, neg)
    idx_ref[0] = acc


def _knn(q_pos, q_post):
    b, n, _ = q_pos.shape
    return pl.pallas_call(
        _knn_body,
        grid=(b, n // _T2),
        in_specs=[
            pl.BlockSpec((1, 3, n), lambda bi, i: (bi, 0, 0)),
            pl.BlockSpec((1, _T2, 3), lambda bi, i: (bi, i, 0)),
        ],
        out_specs=pl.BlockSpec((1, _T2, _K), lambda bi, i: (bi, i, 0)),
        out_shape=jax.ShapeDtypeStruct((b, n, _K), jnp.int32),
    )(q_post, q_pos)


def _sc_gather(qtab, gidx2d):
    rows = gidx2d.shape[0] * gidx2d.shape[1]
    dcol = qtab.shape[1]
    nw = 32
    per_w = rows // nw
    ch = 128
    nch = per_w // ch
    mesh = plsc.VectorSubcoreMesh(core_axis_name="c", subcore_axis_name="s")

    @functools.partial(
        pl.kernel,
        out_type=jax.ShapeDtypeStruct((rows, dcol), jnp.float32),
        mesh=mesh,
        scratch_types=[
            pltpu.VMEM((nch, ch), jnp.int32),
            pltpu.VMEM((ch, dcol), jnp.float32),
            pltpu.VMEM((ch, dcol), jnp.float32),
            pltpu.SemaphoreType.DMA,
            pltpu.SemaphoreType.DMA,
        ],
    )
    def gk(qtab_hbm, gidx_hbm, out_hbm, idx_v, buf0, buf1, sem0, sem1):
        wid = lax.axis_index("s") * 2 + lax.axis_index("c")
        pltpu.sync_copy(gidx_hbm.at[pl.ds(wid * nch, nch)], idx_v)
        bufs = (buf0, buf1)
        sems = (sem0, sem1)
        cps = [None, None]
        cps[0] = pltpu.async_copy(qtab_hbm.at[idx_v.at[0]], buf0, sem0)
        for i in range(nch):
            cur = i % 2
            if i + 1 < nch:
                cps[1 - cur] = pltpu.async_copy(
                    qtab_hbm.at[idx_v.at[i + 1]], bufs[1 - cur], sems[1 - cur])
            cps[cur].wait()
            pltpu.sync_copy(bufs[cur], out_hbm.at[pl.ds((wid * nch + i) * ch, ch)])

    return gk(qtab, gidx2d)


def _main_body(qg_ref, qt_ref, wk_ref, wa_ref, wp_ref, o_ref):
    t = qt_ref.shape[1]
    c = wa_ref.shape[0]
    r = t * _K
    h = []
    for d in range(3):
        lv = qg_ref[0, :, d, :]
        qrep = jnp.broadcast_to(
            qt_ref[0, :, d, :][:, None, :], (t, _K, c)).reshape(r, c)
        fe = jnp.concatenate([lv - qrep, qrep], axis=1)
        h.append(_mm(fe, wk_ref[...]))
    dm = [_mm(h[d], wa_ref[...]) for d in range(3)]
    dnorm = jnp.sqrt(dm[0] * dm[0] + dm[1] * dm[1] + dm[2] * dm[2]) + _EPS
    inv = 1.0 / dnorm
    ddir = [dm[d] * inv for d in range(3)]
    dot = h[0] * ddir[0] + h[1] * ddir[1] + h[2] * ddir[2]
    acted = jnp.where(dot >= 0, dot, 0.2 * dot)
    ha = [h[d] + (acted - dot) * ddir[d] for d in range(3)]
    dp = [_mm(ha[d], wp_ref[...]) for d in range(3)]
    dotp = (ha[0] * dp[0] + ha[1] * dp[1] + ha[2] * dp[2]).reshape(t, _K, c)
    m = jnp.max(dotp, axis=1)
    ha3 = [x.reshape(t, _K, c) for x in ha]
    outd = [x[:, _K - 1, :] for x in ha3]
    for k in range(_K - 2, -1, -1):
        sel = dotp[:, k, :] == m
        outd = [jnp.where(sel, ha3[d][:, k, :], outd[d]) for d in range(3)]
    o_ref[0] = jnp.stack(outd, axis=1)


def _main(qg, qt, wk, wa, wp):
    b, nk, _, c = qg.shape
    n = qt.shape[1]
    r = _T * _K
    return pl.pallas_call(
        _main_body,
        grid=(b, n // _T),
        in_specs=[
            pl.BlockSpec((1, r, 3, c), lambda bi, i: (bi, i, 0, 0)),
            pl.BlockSpec((1, _T, 3, c), lambda bi, i: (bi, i, 0, 0)),
            pl.BlockSpec((c, 2 * c), lambda bi, i: (0, 0)),
            pl.BlockSpec((c, c), lambda bi, i: (0, 0)),
            pl.BlockSpec((c, c), lambda bi, i: (0, 0)),
        ],
        out_specs=pl.BlockSpec((1, _T, 3, c), lambda bi, i: (bi, i, 0, 0)),
        out_shape=jax.ShapeDtypeStruct((b, n, 3, c), jnp.float32),
    )(qg, qt, wk, wa, wp)


def kernel(q, q_pos, W_knn, W_act, W_pool):
    b, n, c, _ = q.shape
    qtab = jnp.transpose(q, (0, 1, 3, 2))
    q_post = jnp.transpose(q_pos, (0, 2, 1))
    gidx = _knn(q_pos, q_post)
    qg = _sc_gather(qtab.reshape(b * n, 3 * c),
                    gidx.reshape(b * n * _K // 128, 128))
    out = _main(qg.reshape(b, n * _K, 3, c), qtab, W_knn, W_act, W_pool)
    return jnp.transpose(out, (0, 1, 3, 2))


# speedup vs baseline: 1.1737x; 1.0031x over previous
import functools

import jax
import jax.numpy as jnp
from jax import lax
from jax.experimental import pallas as pl
from jax.experimental.pallas import tpu as pltpu
from jax.experimental.pallas import tpu_sc as plsc

_EPS = 1e-6
_K = 16
_T2 = 256
_T = 128

_DN = (((1,), (1,)), ((), ()))


def _mm(x, w):
    return lax.dot_general(x, w, _DN, preferred_element_type=jnp.float32)


def _knn_body(qpt_ref, qp_ref, idx_ref):
    b = pl.program_id(0)
    n = qpt_ref.shape[2]
    t = qp_ref.shape[1]
    qa = qpt_ref[0]
    qt = qp_ref[0]
    mm = lax.dot_general(qt, qa, (((1,), (0,)), ((), ())),
                         preferred_element_type=jnp.float32)
    sqx = jnp.sum(qt * qt, axis=1, keepdims=True)
    sqy = jnp.sum(qa * qa, axis=0, keepdims=True)
    neg = -((-2.0) * mm + sqx + sqy)
    lane = lax.broadcasted_iota(jnp.int32, (t, n), 1)
    kio = lax.broadcasted_iota(jnp.int32, (t, _K), 1)
    acc = jnp.zeros((t, _K), jnp.int32)
    for j in range(_K):
        m = jnp.max(neg, axis=1, keepdims=True)
        cand = jnp.where(neg == m, lane, n)
        am = jnp.min(cand, axis=1, keepdims=True)
        acc = jnp.where(kio == j, am + b * n, acc)
        neg = jnp.where(lane == am, -jnp.inf, neg)
    idx_ref[0] = acc


def _knn(q_pos, q_post):
    b, n, _ = q_pos.shape
    return pl.pallas_call(
        _knn_body,
        grid=(b, n // _T2),
        in_specs=[
            pl.BlockSpec((1, 3, n), lambda bi, i: (bi, 0, 0)),
            pl.BlockSpec((1, _T2, 3), lambda bi, i: (bi, i, 0)),
        ],
        out_specs=pl.BlockSpec((1, _T2, _K), lambda bi, i: (bi, i, 0)),
        out_shape=jax.ShapeDtypeStruct((b, n, _K), jnp.int32),
    )(q_post, q_pos)


def _sc_gather(qtab, gidx2d):
    rows = gidx2d.shape[0] * gidx2d.shape[1]
    dcol = qtab.shape[1]
    nw = 32
    per_w = rows // nw
    ch = 128
    nch = per_w // ch
    mesh = plsc.VectorSubcoreMesh(core_axis_name="c", subcore_axis_name="s")

    @functools.partial(
        pl.kernel,
        out_type=jax.ShapeDtypeStruct((rows, dcol), jnp.float32),
        mesh=mesh,
        compiler_params=pltpu.CompilerParams(use_tc_tiling_on_sc=True),
        scratch_types=[
            pltpu.VMEM((nch, ch), jnp.int32),
            pltpu.VMEM((ch, dcol), jnp.float32),
            pltpu.VMEM((ch, dcol), jnp.float32),
            pltpu.SemaphoreType.DMA,
            pltpu.SemaphoreType.DMA,
        ],
    )
    def gk(qtab_hbm, gidx_hbm, out_hbm, idx_v, buf0, buf1, sem0, sem1):
        wid = lax.axis_index("s") * 2 + lax.axis_index("c")
        pltpu.sync_copy(gidx_hbm.at[pl.ds(wid * nch, nch)], idx_v)
        bufs = (buf0, buf1)
        sems = (sem0, sem1)
        cps = [None, None]
        cps[0] = pltpu.async_copy(qtab_hbm.at[idx_v.at[0]], buf0, sem0)
        for i in range(nch):
            cur = i % 2
            if i + 1 < nch:
                cps[1 - cur] = pltpu.async_copy(
                    qtab_hbm.at[idx_v.at[i + 1]], bufs[1 - cur], sems[1 - cur])
            cps[cur].wait()
            pltpu.sync_copy(bufs[cur], out_hbm.at[pl.ds((wid * nch + i) * ch, ch)])

    return gk(qtab, gidx2d)


def _main_body(qg_ref, qt_ref, wk_ref, wa_ref, wp_ref, o_ref):
    t = qt_ref.shape[1]
    c = wa_ref.shape[0]
    r = t * _K
    h = []
    for d in range(3):
        lv = qg_ref[0, :, d, :]
        qrep = jnp.broadcast_to(
            qt_ref[0, :, d, :][:, None, :], (t, _K, c)).reshape(r, c)
        fe = jnp.concatenate([lv - qrep, qrep], axis=1)
        h.append(_mm(fe, wk_ref[...]))
    dm = [_mm(h[d], wa_ref[...]) for d in range(3)]
    dnorm = jnp.sqrt(dm[0] * dm[0] + dm[1] * dm[1] + dm[2] * dm[2]) + _EPS
    inv = 1.0 / dnorm
    ddir = [dm[d] * inv for d in range(3)]
    dot = h[0] * ddir[0] + h[1] * ddir[1] + h[2] * ddir[2]
    acted = jnp.where(dot >= 0, dot, 0.2 * dot)
    ha = [h[d] + (acted - dot) * ddir[d] for d in range(3)]
    dp = [_mm(ha[d], wp_ref[...]) for d in range(3)]
    dotp = (ha[0] * dp[0] + ha[1] * dp[1] + ha[2] * dp[2]).reshape(t, _K, c)
    m = jnp.max(dotp, axis=1)
    ha3 = [x.reshape(t, _K, c) for x in ha]
    outd = [x[:, _K - 1, :] for x in ha3]
    for k in range(_K - 2, -1, -1):
        sel = dotp[:, k, :] == m
        outd = [jnp.where(sel, ha3[d][:, k, :], outd[d]) for d in range(3)]
    o_ref[0] = jnp.stack(outd, axis=1)


def _main(qg, qt, wk, wa, wp):
    b, nk, _, c = qg.shape
    n = qt.shape[1]
    r = _T * _K
    return pl.pallas_call(
        _main_body,
        grid=(b, n // _T),
        in_specs=[
            pl.BlockSpec((1, r, 3, c), lambda bi, i: (bi, i, 0, 0)),
            pl.BlockSpec((1, _T, 3, c), lambda bi, i: (bi, i, 0, 0)),
            pl.BlockSpec((c, 2 * c), lambda bi, i: (0, 0)),
            pl.BlockSpec((c, c), lambda bi, i: (0, 0)),
            pl.BlockSpec((c, c), lambda bi, i: (0, 0)),
        ],
        out_specs=pl.BlockSpec((1, _T, 3, c), lambda bi, i: (bi, i, 0, 0)),
        out_shape=jax.ShapeDtypeStruct((b, n, 3, c), jnp.float32),
    )(qg, qt, wk, wa, wp)


def kernel(q, q_pos, W_knn, W_act, W_pool):
    b, n, c, _ = q.shape
    qtab = jnp.transpose(q, (0, 1, 3, 2))
    q_post = jnp.transpose(q_pos, (0, 2, 1))
    gidx = _knn(q_pos, q_post)
    qg = _sc_gather(qtab.reshape(b * n, 3 * c),
                    gidx.reshape(b * n * _K // 128, 128))
    out = _main(qg.reshape(b, n * _K, 3, c), qtab, W_knn, W_act, W_pool)
    return jnp.transpose(out, (0, 1, 3, 2))
